# Initial kernel scaffold; baseline (speedup 1.0000x reference)
import functools

import jax
import jax.numpy as jnp
from jax import lax
from jax.experimental import pallas as pl
from jax.experimental.pallas import tpu as pltpu
from jax.experimental.pallas import tpu_sc as plsc

NC = 2
NS = 16
NW = NC * NS
L = 16

D = 128
C = 128
CHUNKS = 123
BPW = C * CHUNKS
E_PAD = BPW * NW


def _sc_body(user_hbm, item_hbm, src_hbm, dst_hbm, out_hbm,
             src_v, dst_v, u_rows, i_rows, sc_v, sem_u, sem_i):
    wid = lax.axis_index("s") * NC + lax.axis_index("c")
    base = wid * BPW
    pltpu.sync_copy(src_hbm.at[pl.ds(base, BPW)], src_v)
    pltpu.sync_copy(dst_hbm.at[pl.ds(base, BPW)], dst_v)

    def chunk_body(k, _):
        off = k * C
        cu = pltpu.async_copy(user_hbm.at[src_v.at[pl.ds(off, C)]], u_rows,
                              sem_u)
        ci = pltpu.async_copy(item_hbm.at[dst_v.at[pl.ds(off, C)]], i_rows,
                              sem_i)
        cu.wait()
        ci.wait()

        def edge_body(e, _):
            acc = jnp.zeros((L,), jnp.float32)
            for t in range(D // L):
                acc = acc + u_rows[e, pl.ds(t * L, L)] * i_rows[e, pl.ds(t * L, L)]
            sc_v[e] = jnp.sum(acc)
            return ()

        lax.fori_loop(0, C, edge_body, (), unroll=False)
        pltpu.sync_copy(sc_v, out_hbm.at[pl.ds(base + off, C)])
        return ()

    lax.fori_loop(0, CHUNKS, chunk_body, (), unroll=False)


@jax.jit
def _sc_scores(user_table, item_table, src, dst):
    mesh = plsc.VectorSubcoreMesh(core_axis_name="c", subcore_axis_name="s")
    return pl.kernel(
        _sc_body,
        out_type=jax.ShapeDtypeStruct((E_PAD,), jnp.float32),
        mesh=mesh,
        scratch_types=[
            pltpu.VMEM((BPW,), jnp.int32),
            pltpu.VMEM((BPW,), jnp.int32),
            pltpu.VMEM((C, D), jnp.float32),
            pltpu.VMEM((C, D), jnp.float32),
            pltpu.VMEM((C,), jnp.float32),
            pltpu.SemaphoreType.DMA,
            pltpu.SemaphoreType.DMA,
        ],
    )(user_table, item_table, src, dst)


def kernel(user_table, item_table, edge_label_index):
    e = edge_label_index.shape[1]
    pad = E_PAD - e
    src = jnp.concatenate([edge_label_index[0],
                           jnp.zeros((pad,), jnp.int32)])
    dst = jnp.concatenate([edge_label_index[1],
                           jnp.zeros((pad,), jnp.int32)])
    scores = _sc_scores(user_table, item_table, src, dst)
    return scores[:e]


# speedup vs baseline: 2.3847x; 2.3847x over previous
import functools

import jax
import jax.numpy as jnp
from jax import lax
from jax.experimental import pallas as pl
from jax.experimental.pallas import tpu as pltpu
from jax.experimental.pallas import tpu_sc as plsc

NC = 2
NS = 16
NW = NC * NS
L = 16

D = 128
C = 128
CHUNKS = 123
BPW = C * CHUNKS
E_PAD = BPW * NW


def _sc_body(user_hbm, item_hbm, src_hbm, dst_hbm, out_hbm,
             src_v, dst_v, u_rows, i_rows, sc_v, sem_u, sem_i):
    wid = lax.axis_index("s") * NC + lax.axis_index("c")
    base = wid * BPW
    pltpu.sync_copy(src_hbm.at[pl.ds(base, BPW)], src_v)
    pltpu.sync_copy(dst_hbm.at[pl.ds(base, BPW)], dst_v)

    def chunk_body(k, _):
        off = k * C
        cu = pltpu.async_copy(user_hbm.at[src_v.at[pl.ds(off, C)]], u_rows,
                              sem_u)
        ci = pltpu.async_copy(item_hbm.at[dst_v.at[pl.ds(off, C)]], i_rows,
                              sem_i)
        cu.wait()
        ci.wait()

        lane = lax.iota(jnp.int32, L)

        def group_body(g, _):
            e0 = g * L
            scores = jnp.zeros((L,), jnp.float32)
            for j in range(L):
                acc = u_rows[e0 + j, pl.ds(0, L)] * i_rows[e0 + j, pl.ds(0, L)]
                for t in range(1, D // L):
                    acc = acc + (u_rows[e0 + j, pl.ds(t * L, L)]
                                 * i_rows[e0 + j, pl.ds(t * L, L)])
                scores = jnp.where(lane == j, jnp.sum(acc), scores)
            sc_v[pl.ds(e0, L)] = scores
            return ()

        lax.fori_loop(0, C // L, group_body, (), unroll=False)
        pltpu.sync_copy(sc_v, out_hbm.at[pl.ds(base + off, C)])
        return ()

    lax.fori_loop(0, CHUNKS, chunk_body, (), unroll=False)


@jax.jit
def _sc_scores(user_table, item_table, src, dst):
    mesh = plsc.VectorSubcoreMesh(core_axis_name="c", subcore_axis_name="s")
    return pl.kernel(
        _sc_body,
        out_type=jax.ShapeDtypeStruct((E_PAD,), jnp.float32),
        mesh=mesh,
        compiler_params=pltpu.CompilerParams(needs_layout_passes=False),
        scratch_types=[
            pltpu.VMEM((BPW,), jnp.int32),
            pltpu.VMEM((BPW,), jnp.int32),
            pltpu.VMEM((C, D), jnp.float32),
            pltpu.VMEM((C, D), jnp.float32),
            pltpu.VMEM((C,), jnp.float32),
            pltpu.SemaphoreType.DMA,
            pltpu.SemaphoreType.DMA,
        ],
    )(user_table, item_table, src, dst)


def kernel(user_table, item_table, edge_label_index):
    e = edge_label_index.shape[1]
    pad = E_PAD - e
    src = jnp.concatenate([edge_label_index[0],
                           jnp.zeros((pad,), jnp.int32)])
    dst = jnp.concatenate([edge_label_index[1],
                           jnp.zeros((pad,), jnp.int32)])
    scores = _sc_scores(user_table, item_table, src, dst)
    return scores[:e]


# speedup vs baseline: 3.3656x; 1.4113x over previous
import functools

import jax
import jax.numpy as jnp
from jax import lax
from jax.experimental import pallas as pl
from jax.experimental.pallas import tpu as pltpu
from jax.experimental.pallas import tpu_sc as plsc

NC = 2
NS = 16
NW = NC * NS
L = 16

D = 128
C = 128
CHUNKS = 124
BPW = C * CHUNKS
E_PAD = BPW * NW
TSTRIDE = L + 1


def _sc_body(user_hbm, item_hbm, src_hbm, dst_hbm, out_hbm,
             src_v, dst_v, u0, i0, u1, i1, sc_all, tmp,
             sem_u0, sem_i0, sem_u1, sem_i1):
    wid = lax.axis_index("s") * NC + lax.axis_index("c")
    base = wid * BPW
    pltpu.sync_copy(src_hbm.at[pl.ds(base, BPW)], src_v)
    pltpu.sync_copy(dst_hbm.at[pl.ds(base, BPW)], dst_v)

    bufs = ((u0, i0, sem_u0, sem_i0), (u1, i1, sem_u1, sem_i1))

    def issue(k, slot):
        u_r, i_r, s_u, s_i = bufs[slot]
        pltpu.async_copy(user_hbm.at[src_v.at[pl.ds(k * C, C)]], u_r, s_u)
        pltpu.async_copy(item_hbm.at[dst_v.at[pl.ds(k * C, C)]], i_r, s_i)

    lane = lax.iota(jnp.int32, L)
    col0 = lane * TSTRIDE

    def compute(k, slot):
        u_r, i_r, s_u, s_i = bufs[slot]
        pltpu.make_async_copy(user_hbm.at[src_v.at[pl.ds(k * C, C)]],
                              u_r, s_u).wait()
        pltpu.make_async_copy(item_hbm.at[dst_v.at[pl.ds(k * C, C)]],
                              i_r, s_i).wait()

        def group_body(g, _):
            e0 = g * L
            for j in range(L):
                acc = u_r[e0 + j, pl.ds(0, L)] * i_r[e0 + j, pl.ds(0, L)]
                for t in range(1, D // L):
                    acc = acc + (u_r[e0 + j, pl.ds(t * L, L)]
                                 * i_r[e0 + j, pl.ds(t * L, L)])
                tmp[pl.ds(j * TSTRIDE, L)] = acc
            scores = plsc.load_gather(tmp, [col0])
            for k2 in range(1, L):
                scores = scores + plsc.load_gather(tmp, [col0 + k2])
            sc_all[pl.ds(k * C + e0, L)] = scores
            return ()

        lax.fori_loop(0, C // L, group_body, (), unroll=False)

    issue(0, 0)

    def body2(k2, _):
        k = k2 * 2
        issue(k + 1, 1)
        compute(k, 0)

        @pl.when(k + 2 < CHUNKS)
        def _():
            issue(k + 2, 0)

        compute(k + 1, 1)
        return ()

    lax.fori_loop(0, CHUNKS // 2, body2, (), unroll=False)
    pltpu.sync_copy(sc_all, out_hbm.at[pl.ds(base, BPW)])


@jax.jit
def _sc_scores(user_table, item_table, src, dst):
    mesh = plsc.VectorSubcoreMesh(core_axis_name="c", subcore_axis_name="s")
    return pl.kernel(
        _sc_body,
        out_type=jax.ShapeDtypeStruct((E_PAD,), jnp.float32),
        mesh=mesh,
        compiler_params=pltpu.CompilerParams(needs_layout_passes=False),
        scratch_types=[
            pltpu.VMEM((BPW,), jnp.int32),
            pltpu.VMEM((BPW,), jnp.int32),
            pltpu.VMEM((C, D), jnp.float32),
            pltpu.VMEM((C, D), jnp.float32),
            pltpu.VMEM((C, D), jnp.float32),
            pltpu.VMEM((C, D), jnp.float32),
            pltpu.VMEM((BPW,), jnp.float32),
            pltpu.VMEM((L * TSTRIDE,), jnp.float32),
            pltpu.SemaphoreType.DMA,
            pltpu.SemaphoreType.DMA,
            pltpu.SemaphoreType.DMA,
            pltpu.SemaphoreType.DMA,
        ],
    )(user_table, item_table, src, dst)


def kernel(user_table, item_table, edge_label_index):
    e = edge_label_index.shape[1]
    pad = E_PAD - e
    src = jnp.concatenate([edge_label_index[0],
                           jnp.zeros((pad,), jnp.int32)])
    dst = jnp.concatenate([edge_label_index[1],
                           jnp.zeros((pad,), jnp.int32)])
    scores = _sc_scores(user_table, item_table, src, dst)
    return scores[:e]


# speedup vs baseline: 7.1131x; 2.1135x over previous
import functools

import jax
import jax.numpy as jnp
from jax import lax
from jax.experimental import pallas as pl
from jax.experimental.pallas import tpu as pltpu
from jax.experimental.pallas import tpu_sc as plsc

NC = 2
NS = 16
NW = NC * NS
L = 16

D = 128
C = 64
NBUF = 4
CHUNKS = 248
BPW = C * CHUNKS
E_PAD = BPW * NW
TSTRIDE = L + 1


def _sc_body(user_hbm, item_hbm, src_hbm, dst_hbm, out_hbm,
             src_v, dst_v, u_bufs, i_bufs, sc_all, tmp, sem_u, sem_i):
    wid = lax.axis_index("s") * NC + lax.axis_index("c")
    base = wid * BPW
    pltpu.sync_copy(src_hbm.at[pl.ds(base, BPW)], src_v)
    pltpu.sync_copy(dst_hbm.at[pl.ds(base, BPW)], dst_v)

    def issue(k, b):
        pltpu.async_copy(user_hbm.at[src_v.at[pl.ds(k * C, C)]],
                         u_bufs[b], sem_u[b])
        pltpu.async_copy(item_hbm.at[dst_v.at[pl.ds(k * C, C)]],
                         i_bufs[b], sem_i[b])

    lane = lax.iota(jnp.int32, L)
    col0 = lane * TSTRIDE

    def compute(k, b):
        u_r = u_bufs[b]
        i_r = i_bufs[b]
        pltpu.make_async_copy(user_hbm.at[src_v.at[pl.ds(k * C, C)]],
                              u_r, sem_u[b]).wait()
        pltpu.make_async_copy(item_hbm.at[dst_v.at[pl.ds(k * C, C)]],
                              i_r, sem_i[b]).wait()

        def group_body(g, _):
            e0 = g * L
            for j in range(L):
                acc = u_r[e0 + j, pl.ds(0, L)] * i_r[e0 + j, pl.ds(0, L)]
                for t in range(1, D // L):
                    acc = acc + (u_r[e0 + j, pl.ds(t * L, L)]
                                 * i_r[e0 + j, pl.ds(t * L, L)])
                tmp[pl.ds(j * TSTRIDE, L)] = acc
            scores = plsc.load_gather(tmp, [col0])
            for k2 in range(1, L):
                scores = scores + plsc.load_gather(tmp, [col0 + k2])
            sc_all[pl.ds(k * C + e0, L)] = scores
            return ()

        lax.fori_loop(0, C // L, group_body, (), unroll=False)

    for b in range(NBUF):
        issue(b, b)

    Q = CHUNKS // NBUF

    def body(q, _):
        k0 = q * NBUF
        for b in range(NBUF):
            compute(k0 + b, b)

            @pl.when(k0 + b + NBUF < CHUNKS)
            def _():
                issue(k0 + b + NBUF, b)
        return ()

    lax.fori_loop(0, Q, body, (), unroll=False)
    pltpu.sync_copy(sc_all, out_hbm.at[pl.ds(base, BPW)])


@jax.jit
def _sc_scores(user_table, item_table, src, dst):
    mesh = plsc.VectorSubcoreMesh(core_axis_name="c", subcore_axis_name="s")
    return pl.kernel(
        _sc_body,
        out_type=jax.ShapeDtypeStruct((E_PAD,), jnp.float32),
        mesh=mesh,
        compiler_params=pltpu.CompilerParams(needs_layout_passes=False),
        scratch_types=[
            pltpu.VMEM((BPW,), jnp.int32),
            pltpu.VMEM((BPW,), jnp.int32),
            [pltpu.VMEM((C, D), jnp.float32) for _ in range(NBUF)],
            [pltpu.VMEM((C, D), jnp.float32) for _ in range(NBUF)],
            pltpu.VMEM((BPW,), jnp.float32),
            pltpu.VMEM((L * TSTRIDE,), jnp.float32),
            [pltpu.SemaphoreType.DMA for _ in range(NBUF)],
            [pltpu.SemaphoreType.DMA for _ in range(NBUF)],
        ],
    )(user_table, item_table, src, dst)


def kernel(user_table, item_table, edge_label_index):
    e = edge_label_index.shape[1]
    pad = E_PAD - e
    pad_idx = jnp.arange(pad, dtype=jnp.int32) % user_table.shape[0]
    src = jnp.concatenate([edge_label_index[0], pad_idx])
    dst = jnp.concatenate([edge_label_index[1], pad_idx])
    scores = _sc_scores(user_table, item_table, src, dst)
    return scores[:e]
